# baseline (speedup 1.0000x reference)
import functools

import jax
import jax.numpy as jnp
import numpy as np
from jax import lax
from jax.experimental import pallas as pl
from jax.experimental.pallas import tpu as pltpu
from jax.experimental.pallas import tpu_sc as plsc

NSF = 16
N_NODES = 10000
N_EDGES = 320000
TPW = 80
NORM = 1.0 / np.sqrt(NSF)

NW = 32
PER_TILE = N_EDGES // NW
CH = 80
KF = 5
ROWS_OUT = CH * KF
OUT_ITERS = PER_TILE // ROWS_OUT
NCHUNK = PER_TILE // CH
NPT = N_NODES // 16

BLK = 2000

_F32 = jnp.float32


def _consts():
    p = np.arange(2 * NSF * NSF)
    R = np.zeros((NSF, 2 * NSF * NSF), np.float32)
    R[(p % 256) // NSF, p] = 1.0
    P = np.zeros((2 * NSF * NSF, 2 * NSF), np.float32)
    P[p, np.where(p < 256, 0, NSF) + p % NSF] = NORM
    E16 = np.zeros((NSF, 3 * NSF), np.float32)
    E3 = np.zeros((3, 3 * NSF), np.float32)
    for j in range(NSF):
        for c in range(3):
            E16[j, 3 * j + c] = 1.0
            E3[c, 3 * j + c] = 1.0
    ones = np.zeros((1, 16), np.float32)
    ones[0, 0] = 1.0
    return (jnp.asarray(R), jnp.asarray(P), jnp.asarray(E16), jnp.asarray(E3),
            jnp.asarray(ones))


def _node_mlp_body(x_ref, w1_ref, b1_ref, w2_ref, b2_ref, h_ref):
    t = jnp.maximum(
        jnp.dot(x_ref[...], w1_ref[...], preferred_element_type=_F32)
        + b1_ref[...], 0.0)
    h_ref[...] = (jnp.dot(t, w2_ref[...], preferred_element_type=_F32)
                  + b2_ref[...])


def _node_mlp(x, Wn1, bn1, Wn2, bn2):
    return pl.pallas_call(
        _node_mlp_body,
        out_shape=jax.ShapeDtypeStruct((N_NODES, NSF), _F32),
    )(x, Wn1, bn1.reshape(1, -1), Wn2, bn2.reshape(1, -1))


_MESH = plsc.VectorSubcoreMesh(core_axis_name="c", subcore_axis_name="s")


@functools.partial(
    pl.kernel,
    out_type=(jax.ShapeDtypeStruct((N_EDGES, NSF), _F32),
              jax.ShapeDtypeStruct((N_EDGES, NSF), _F32)),
    mesh=_MESH,
    scratch_types=[
        pltpu.VMEM((NCHUNK, CH), jnp.int32),
        pltpu.VMEM((NCHUNK, CH), jnp.int32),
        pltpu.VMEM((ROWS_OUT, NSF), _F32),
        pltpu.VMEM((ROWS_OUT, NSF), _F32),
        pltpu.SemaphoreType.DMA,
        pltpu.SemaphoreType.DMA,
    ],
    compiler_params=pltpu.CompilerParams(use_tc_tiling_on_sc=False),
)
def _sc_gather(h_hbm, src_hbm, dst_hbm, hs_hbm, hd_hbm,
               idx_s, idx_d, rows_s, rows_d, sem_s, sem_d):
    c = lax.axis_index("c")
    s = lax.axis_index("s")
    wid = s * 2 + c
    base = wid * PER_TILE
    pltpu.sync_copy(src_hbm.at[wid], idx_s)
    pltpu.sync_copy(dst_hbm.at[wid], idx_d)

    def outer(j, carry):
        cps = []
        for b in range(KF):
            k = j * KF + b
            cps.append(pltpu.async_copy(
                h_hbm.at[idx_s.at[k]], rows_s.at[pl.ds(b * CH, CH)], sem_s))
            cps.append(pltpu.async_copy(
                h_hbm.at[idx_d.at[k]], rows_d.at[pl.ds(b * CH, CH)], sem_d))
        for cp in cps:
            cp.wait()
        row0 = base + j * ROWS_OUT
        pltpu.sync_copy(rows_s, hs_hbm.at[pl.ds(row0, ROWS_OUT)])
        pltpu.sync_copy(rows_d, hd_hbm.at[pl.ds(row0, ROWS_OUT)])
        return carry

    lax.fori_loop(0, OUT_ITERS, outer, 0)


def _edge_body(ea_ref, sh_ref, hs_ref, hd_ref,
               We1_ref, be1_ref, We2_ref, be2_ref,
               W1e_ref, W1s_ref, W1d_ref, bf1_ref, Wf2_ref, bf2_ref,
               R_ref, P_ref, E16_ref, E3_ref, ones_ref, tp_ref):
    pet = dict(preferred_element_type=_F32)
    t = jnp.maximum(jnp.dot(ea_ref[...], We1_ref[...], **pet) + be1_ref[...],
                    0.0)
    e = jnp.dot(t, We2_ref[...], **pet) + be2_ref[...]
    hs = hs_ref[...]
    g = jnp.maximum(
        jnp.dot(e, W1e_ref[...], **pet)
        + jnp.dot(hs, W1s_ref[...], **pet)
        + jnp.dot(hd_ref[...], W1d_ref[...], **pet)
        + bf1_ref[...], 0.0)
    w = jnp.dot(g.astype(jnp.bfloat16), Wf2_ref[...], **pet) + bf2_ref[...]
    m = w * jnp.dot(hs.astype(jnp.bfloat16), R_ref[...], **pet)
    sred = jnp.dot(m.astype(jnp.bfloat16), P_ref[...], **pet)
    sh = sh_ref[...]
    o0 = sred[:, :NSF] * sh[:, 0:1]
    o1 = (jnp.dot(sred[:, NSF:], E16_ref[...], **pet)
          * jnp.dot(sh[:, 1:4], E3_ref[...], **pet))
    onecol = jnp.broadcast_to(ones_ref[...], (BLK, 16))
    tp_ref[...] = jnp.concatenate([o0, o1, onecol], axis=-1)


def _edge_compute(edge_attr, edge_sh, hs, hd, We1, be1, We2, be2,
                  Wf1, bf1, Wf2, bf2, R, P, E16, E3, ones):
    grid = (N_EDGES // BLK,)
    blk = lambda r, c: pl.BlockSpec((r, c), lambda i: (i, 0))
    full = lambda a: pl.BlockSpec(a.shape, lambda i: (0,) * a.ndim)
    args = (edge_attr, edge_sh, hs, hd,
            We1, be1.reshape(1, -1), We2, be2.reshape(1, -1),
            Wf1[:NSF], Wf1[NSF:2 * NSF], Wf1[2 * NSF:],
            bf1.reshape(1, -1), Wf2.astype(jnp.bfloat16), bf2.reshape(1, -1),
            R.astype(jnp.bfloat16), P.astype(jnp.bfloat16), E16, E3, ones)
    in_specs = [blk(BLK, edge_attr.shape[1]), blk(BLK, 9),
                blk(BLK, NSF), blk(BLK, NSF)] + [full(a) for a in args[4:]]
    return pl.pallas_call(
        _edge_body,
        grid=grid,
        in_specs=in_specs,
        out_specs=blk(BLK, TPW),
        out_shape=jax.ShapeDtypeStruct((N_EDGES, TPW), _F32),
    )(*args)


@functools.partial(
    pl.kernel,
    out_type=jax.ShapeDtypeStruct((2 * N_NODES, TPW), _F32),
    mesh=_MESH,
    scratch_types=[
        pltpu.VMEM((NCHUNK, CH), jnp.int32),
        pltpu.VMEM((ROWS_OUT, TPW), _F32),
        pltpu.VMEM_SHARED((N_NODES, TPW), _F32),
    ],
    compiler_params=pltpu.CompilerParams(use_tc_tiling_on_sc=False),
)
def _sc_scatter(tp_hbm, dst_hbm, zero_hbm, part_hbm, idx_d, rows, acc):
    c = lax.axis_index("c")
    s = lax.axis_index("s")
    wid = s * 2 + c
    base = wid * PER_TILE
    pltpu.sync_copy(zero_hbm.at[pl.ds(s * NPT, NPT)], acc.at[pl.ds(s * NPT, NPT)])
    pltpu.sync_copy(dst_hbm.at[wid], idx_d)
    plsc.subcore_barrier()

    def outer(j, carry):
        pltpu.sync_copy(tp_hbm.at[pl.ds(base + j * ROWS_OUT, ROWS_OUT)], rows)
        for b in range(KF):
            pltpu.sync_copy(rows.at[pl.ds(b * CH, CH)],
                            acc.at[idx_d.at[j * KF + b]], add=True)
        return carry

    lax.fori_loop(0, OUT_ITERS, outer, 0)
    plsc.subcore_barrier()
    pltpu.sync_copy(acc.at[pl.ds(s * NPT, NPT)],
                    part_hbm.at[pl.ds(c * N_NODES + s * NPT, NPT)])


def _combine_body(p_ref, out_ref):
    p = p_ref[0] + p_ref[1]
    cnt = p[:, 4 * NSF:4 * NSF + 1]
    out_ref[...] = p[:, :4 * NSF] / jnp.maximum(cnt, 1.0)


def _combine(parts):
    return pl.pallas_call(
        _combine_body,
        out_shape=jax.ShapeDtypeStruct((N_NODES, 4 * NSF), _F32),
    )(parts.reshape(2, N_NODES, TPW))


def kernel(x, edge_index, edge_attr, edge_sh, Wn1, bn1, Wn2, bn2,
           We1, be1, We2, be2, Wf1, bf1, Wf2, bf2):
    src = edge_index[0].reshape(NW, NCHUNK, CH)
    dst = edge_index[1].reshape(NW, NCHUNK, CH)
    R, P, E16, E3, ones = _consts()

    h = _node_mlp(x, Wn1, bn1, Wn2, bn2)
    hs, hd = _sc_gather(h, src, dst)
    tp = _edge_compute(edge_attr, edge_sh, hs, hd, We1, be1, We2, be2,
                       Wf1, bf1, Wf2, bf2, R, P, E16, E3, ones)
    zero = jnp.zeros((N_NODES, TPW), _F32)
    parts = _sc_scatter(tp, dst, zero)
    return _combine(parts)


# speedup vs baseline: 1.0845x; 1.0845x over previous
import functools

import jax
import jax.numpy as jnp
import numpy as np
from jax import lax
from jax.experimental import pallas as pl
from jax.experimental.pallas import tpu as pltpu
from jax.experimental.pallas import tpu_sc as plsc

NSF = 16
N_NODES = 10000
N_EDGES = 320000
TPW = 128
ACC_W = 80
NORM = 1.0 / np.sqrt(NSF)

NW = 32
PER_TILE = N_EDGES // NW
CH = 80
KF = 5
ROWS_OUT = CH * KF
OUT_ITERS = PER_TILE // ROWS_OUT
NCHUNK = PER_TILE // CH
NPT = N_NODES // 16

BLK = 2000

_F32 = jnp.float32


def _consts():
    p = np.arange(2 * NSF * NSF)
    R = np.zeros((NSF, 2 * NSF * NSF), np.float32)
    R[(p % 256) // NSF, p] = 1.0
    P = np.zeros((2 * NSF * NSF, 2 * NSF), np.float32)
    P[p, np.where(p < 256, 0, NSF) + p % NSF] = NORM
    E16 = np.zeros((NSF, 3 * NSF), np.float32)
    E3 = np.zeros((3, 3 * NSF), np.float32)
    for j in range(NSF):
        for c in range(3):
            E16[j, 3 * j + c] = 1.0
            E3[c, 3 * j + c] = 1.0
    ones = np.zeros((1, TPW - 4 * NSF), np.float32)
    ones[0, 0] = 1.0
    return (jnp.asarray(R), jnp.asarray(P), jnp.asarray(E16), jnp.asarray(E3),
            jnp.asarray(ones))


def _node_mlp_body(x_ref, w1_ref, b1_ref, w2_ref, b2_ref, h_ref):
    t = jnp.maximum(
        jnp.dot(x_ref[...], w1_ref[...], preferred_element_type=_F32)
        + b1_ref[...], 0.0)
    h_ref[...] = (jnp.dot(t, w2_ref[...], preferred_element_type=_F32)
                  + b2_ref[...])


def _node_mlp(x, Wn1, bn1, Wn2, bn2):
    return pl.pallas_call(
        _node_mlp_body,
        out_shape=jax.ShapeDtypeStruct((N_NODES, NSF), _F32),
    )(x, Wn1, bn1.reshape(1, -1), Wn2, bn2.reshape(1, -1))


_MESH = plsc.VectorSubcoreMesh(core_axis_name="c", subcore_axis_name="s")


@functools.partial(
    pl.kernel,
    out_type=(jax.ShapeDtypeStruct((N_EDGES, NSF), _F32),
              jax.ShapeDtypeStruct((N_EDGES, NSF), _F32)),
    mesh=_MESH,
    scratch_types=[
        pltpu.VMEM((NCHUNK, CH), jnp.int32),
        pltpu.VMEM((NCHUNK, CH), jnp.int32),
        pltpu.VMEM((ROWS_OUT, NSF), _F32),
        pltpu.VMEM((ROWS_OUT, NSF), _F32),
        pltpu.SemaphoreType.DMA,
        pltpu.SemaphoreType.DMA,
    ],
    compiler_params=pltpu.CompilerParams(use_tc_tiling_on_sc=False),
)
def _sc_gather(h_hbm, src_hbm, dst_hbm, hs_hbm, hd_hbm,
               idx_s, idx_d, rows_s, rows_d, sem_s, sem_d):
    c = lax.axis_index("c")
    s = lax.axis_index("s")
    wid = s * 2 + c
    base = wid * PER_TILE
    pltpu.sync_copy(src_hbm.at[wid], idx_s)
    pltpu.sync_copy(dst_hbm.at[wid], idx_d)

    def outer(j, carry):
        cps = []
        for b in range(KF):
            k = j * KF + b
            cps.append(pltpu.async_copy(
                h_hbm.at[idx_s.at[k]], rows_s.at[pl.ds(b * CH, CH)], sem_s))
            cps.append(pltpu.async_copy(
                h_hbm.at[idx_d.at[k]], rows_d.at[pl.ds(b * CH, CH)], sem_d))
        for cp in cps:
            cp.wait()
        row0 = base + j * ROWS_OUT
        pltpu.sync_copy(rows_s, hs_hbm.at[pl.ds(row0, ROWS_OUT)])
        pltpu.sync_copy(rows_d, hd_hbm.at[pl.ds(row0, ROWS_OUT)])
        return carry

    lax.fori_loop(0, OUT_ITERS, outer, 0)


def _edge_body(ea_ref, sh_ref, hs_ref, hd_ref,
               We1_ref, be1_ref, We2_ref, be2_ref,
               Wf1_ref, bf1_ref, Wf2_ref, bf2_ref,
               R_ref, P_ref, E16_ref, E3_ref, ones_ref, tp_ref):
    pet = dict(preferred_element_type=_F32)
    bf = jnp.bfloat16
    t = jnp.maximum(
        jnp.dot(ea_ref[...].astype(bf), We1_ref[...], **pet) + be1_ref[...],
        0.0)
    e = jnp.dot(t.astype(bf), We2_ref[...], **pet) + be2_ref[...]
    hs = hs_ref[...]
    cat = jnp.concatenate([e, hs, hd_ref[...]], axis=-1)
    g = jnp.maximum(
        jnp.dot(cat.astype(bf), Wf1_ref[...], **pet) + bf1_ref[...], 0.0)
    w = jnp.dot(g.astype(bf), Wf2_ref[...], **pet) + bf2_ref[...]
    m = w * jnp.dot(hs.astype(bf), R_ref[...], **pet)
    sred = jnp.dot(m.astype(bf), P_ref[...], **pet)
    sh = sh_ref[...]
    o0 = sred[:, :NSF] * sh[:, 0:1]
    o1 = (jnp.dot(sred[:, NSF:].astype(bf), E16_ref[...], **pet)
          * jnp.dot(sh[:, 1:4].astype(bf), E3_ref[...], **pet))
    onecol = jnp.broadcast_to(ones_ref[...], (BLK, TPW - 3 * NSF - NSF))
    tp_ref[...] = jnp.concatenate([o0, o1, onecol], axis=-1)


def _edge_compute(edge_attr, edge_sh, hs, hd, We1, be1, We2, be2,
                  Wf1, bf1, Wf2, bf2, R, P, E16, E3, ones):
    grid = (N_EDGES // BLK,)
    blk = lambda r, c: pl.BlockSpec((r, c), lambda i: (i, 0))
    full = lambda a: pl.BlockSpec(a.shape, lambda i: (0,) * a.ndim)
    bf = jnp.bfloat16
    args = (edge_attr, edge_sh, hs, hd,
            We1.astype(bf), be1.reshape(1, -1), We2.astype(bf),
            be2.reshape(1, -1), Wf1.astype(bf),
            bf1.reshape(1, -1), Wf2.astype(bf), bf2.reshape(1, -1),
            R.astype(bf), P.astype(bf), E16.astype(bf), E3.astype(bf), ones)
    in_specs = [blk(BLK, edge_attr.shape[1]), blk(BLK, 9),
                blk(BLK, NSF), blk(BLK, NSF)] + [full(a) for a in args[4:]]
    return pl.pallas_call(
        _edge_body,
        grid=grid,
        in_specs=in_specs,
        out_specs=blk(BLK, TPW),
        out_shape=jax.ShapeDtypeStruct((N_EDGES, TPW), _F32),
    )(*args)


@functools.partial(
    pl.kernel,
    out_type=jax.ShapeDtypeStruct((2 * N_NODES, ACC_W), _F32),
    mesh=_MESH,
    scratch_types=[
        pltpu.VMEM((NCHUNK, CH), jnp.int32),
        pltpu.VMEM((ROWS_OUT, ACC_W), _F32),
        pltpu.VMEM_SHARED((N_NODES, ACC_W), _F32),
    ],
    compiler_params=pltpu.CompilerParams(use_tc_tiling_on_sc=False),
)
def _sc_scatter(tp_hbm, dst_hbm, zero_hbm, part_hbm, idx_d, rows, acc):
    c = lax.axis_index("c")
    s = lax.axis_index("s")
    wid = s * 2 + c
    base = wid * PER_TILE
    pltpu.sync_copy(zero_hbm.at[pl.ds(s * NPT, NPT)], acc.at[pl.ds(s * NPT, NPT)])
    pltpu.sync_copy(dst_hbm.at[wid], idx_d)
    plsc.subcore_barrier()

    def outer(j, carry):
        pltpu.sync_copy(
            tp_hbm.at[pl.ds(base + j * ROWS_OUT, ROWS_OUT), pl.ds(0, ACC_W)],
            rows)
        for b in range(KF):
            pltpu.sync_copy(rows.at[pl.ds(b * CH, CH)],
                            acc.at[idx_d.at[j * KF + b]], add=True)
        return carry

    lax.fori_loop(0, OUT_ITERS, outer, 0)
    plsc.subcore_barrier()
    pltpu.sync_copy(acc.at[pl.ds(s * NPT, NPT)],
                    part_hbm.at[pl.ds(c * N_NODES + s * NPT, NPT)])


def _combine_body(p_ref, out_ref):
    p = p_ref[0] + p_ref[1]
    cnt = p[:, 4 * NSF:4 * NSF + 1]
    out_ref[...] = p[:, :4 * NSF] / jnp.maximum(cnt, 1.0)


def _combine(parts):
    return pl.pallas_call(
        _combine_body,
        out_shape=jax.ShapeDtypeStruct((N_NODES, 4 * NSF), _F32),
    )(parts.reshape(2, N_NODES, ACC_W))


def kernel(x, edge_index, edge_attr, edge_sh, Wn1, bn1, Wn2, bn2,
           We1, be1, We2, be2, Wf1, bf1, Wf2, bf2):
    src = edge_index[0].reshape(NW, NCHUNK, CH)
    dst = edge_index[1].reshape(NW, NCHUNK, CH)
    R, P, E16, E3, ones = _consts()

    h = _node_mlp(x, Wn1, bn1, Wn2, bn2)
    hs, hd = _sc_gather(h, src, dst)
    tp = _edge_compute(edge_attr, edge_sh, hs, hd, We1, be1, We2, be2,
                       Wf1, bf1, Wf2, bf2, R, P, E16, E3, ones)
    zero = jnp.zeros((N_NODES, ACC_W), _F32)
    parts = _sc_scatter(tp, dst, zero)
    return _combine(parts)


# speedup vs baseline: 1.0876x; 1.0029x over previous
import functools

import jax
import jax.numpy as jnp
import numpy as np
from jax import lax
from jax.experimental import pallas as pl
from jax.experimental.pallas import tpu as pltpu
from jax.experimental.pallas import tpu_sc as plsc

NSF = 16
N_NODES = 10000
N_EDGES = 320000
TPW = 128
ACC_W = 80
NORM = 1.0 / np.sqrt(NSF)

NW = 32
PER_TILE = N_EDGES // NW
CH = 80
KF = 5
ROWS_OUT = CH * KF
OUT_ITERS = PER_TILE // ROWS_OUT
NCHUNK = PER_TILE // CH
NPT = N_NODES // 16


BLK = 3200

_F32 = jnp.float32


def _consts():
    p = np.arange(2 * NSF * NSF)
    R = np.zeros((NSF, 2 * NSF * NSF), np.float32)
    R[(p % 256) // NSF, p] = 1.0
    P = np.zeros((2 * NSF * NSF, 2 * NSF), np.float32)
    P[p, np.where(p < 256, 0, NSF) + p % NSF] = NORM
    E16 = np.zeros((NSF, 3 * NSF), np.float32)
    E3 = np.zeros((3, 3 * NSF), np.float32)
    for j in range(NSF):
        for c in range(3):
            E16[j, 3 * j + c] = 1.0
            E3[c, 3 * j + c] = 1.0
    ones = np.zeros((1, NSF), np.float32)
    ones[0, 0] = 1.0
    return (jnp.asarray(R), jnp.asarray(P), jnp.asarray(E16), jnp.asarray(E3),
            jnp.asarray(ones))


def _node_mlp_body(x_ref, w1_ref, b1_ref, w2_ref, b2_ref, ws_ref, wd_ref,
                   t_ref):
    t = jnp.maximum(
        jnp.dot(x_ref[...], w1_ref[...], preferred_element_type=_F32)
        + b1_ref[...], 0.0)
    h = (jnp.dot(t, w2_ref[...], preferred_element_type=_F32)
         + b2_ref[...])
    a = jnp.dot(h, ws_ref[...], preferred_element_type=_F32)
    b = jnp.dot(h, wd_ref[...], preferred_element_type=_F32)
    t_ref[:N_NODES, :] = jnp.concatenate([a, h], axis=-1)
    t_ref[N_NODES:, :] = jnp.concatenate(
        [b, jnp.zeros((N_NODES, NSF), _F32)], axis=-1)


def _node_mlp(x, Wn1, bn1, Wn2, bn2, Wf1):
    return pl.pallas_call(
        _node_mlp_body,
        out_shape=jax.ShapeDtypeStruct((2 * N_NODES, 4 * NSF), _F32),
    )(x, Wn1, bn1.reshape(1, -1), Wn2, bn2.reshape(1, -1),
      Wf1[NSF:2 * NSF], Wf1[2 * NSF:])


_MESH = plsc.VectorSubcoreMesh(core_axis_name="c", subcore_axis_name="s")


PER_TILE2 = 2 * PER_TILE
NCHUNK2 = PER_TILE2 // CH
ITERS2 = PER_TILE2 // ROWS_OUT


@functools.partial(
    pl.kernel,
    out_type=jax.ShapeDtypeStruct((2 * N_EDGES, 4 * NSF), _F32),
    mesh=_MESH,
    scratch_types=[
        pltpu.VMEM((NCHUNK2, CH), jnp.int32),
        pltpu.VMEM((ROWS_OUT, 4 * NSF), _F32),
        pltpu.SemaphoreType.DMA,
    ],
    compiler_params=pltpu.CompilerParams(use_tc_tiling_on_sc=False),
)
def _sc_gather(tab_hbm, idx_hbm, hsab_hbm, idx_v, rows, sem):
    c = lax.axis_index("c")
    s = lax.axis_index("s")
    wid = s * 2 + c
    base = wid * PER_TILE2
    pltpu.sync_copy(idx_hbm.at[wid], idx_v)

    def outer(j, carry):
        cps = []
        for b in range(KF):
            k = j * KF + b
            cps.append(pltpu.async_copy(
                tab_hbm.at[idx_v.at[k]], rows.at[pl.ds(b * CH, CH)], sem))
        for cp in cps:
            cp.wait()
        pltpu.sync_copy(rows, hsab_hbm.at[pl.ds(base + j * ROWS_OUT, ROWS_OUT)])
        return carry

    lax.fori_loop(0, ITERS2, outer, 0)


def _edge_body(ea_ref, hsab_ref,
               We1_ref, be1_ref, We2_ref, be2_ref,
               bf1_ref, Wf1e_ref, Wf2_ref, bf2_ref,
               R_ref, P_ref, E16_ref, E3_ref, ones_ref, tp_ref):
    pet = dict(preferred_element_type=_F32)
    bf = jnp.bfloat16
    ea = ea_ref[...]
    attr = ea[:, :80]
    sh = ea[:, 80:84]
    hsab = hsab_ref[...]
    a = hsab[:, : 3 * NSF]
    hs = hsab[:, 3 * NSF: 4 * NSF]
    b = hsab[:, 4 * NSF: 7 * NSF]
    t = jnp.maximum(
        jnp.dot(attr.astype(bf), We1_ref[...], **pet) + be1_ref[...], 0.0)
    e = jnp.dot(t.astype(bf), We2_ref[...], **pet) + be2_ref[...]
    g = jnp.maximum(
        jnp.dot(e.astype(bf), Wf1e_ref[...], **pet) + a + b + bf1_ref[...],
        0.0)
    w = (jnp.dot(g.astype(bf), Wf2_ref[...], **pet)
         + bf2_ref[...]).astype(bf)
    m = w * jnp.dot(hs.astype(bf), R_ref[...], **pet).astype(bf)
    sred = jnp.dot(m, P_ref[...], **pet)
    o0 = sred[:, :NSF] * sh[:, 0:1]
    o1 = (jnp.dot(sred[:, NSF:].astype(bf), E16_ref[...], **pet)
          * jnp.dot(sh[:, 1:4].astype(bf), E3_ref[...], **pet))
    tp_ref[:, : 4 * NSF] = jnp.concatenate([o0, o1], axis=-1)
    tp_ref[:, 4 * NSF: 5 * NSF] = jnp.broadcast_to(ones_ref[...], (BLK, NSF))


def _edge_compute(ea128, hsab, We1, be1, We2, be2,
                  Wf1, bf1, Wf2, bf2, R, P, E16, E3, ones):
    grid = (N_EDGES // BLK,)
    blk = lambda r, c: pl.BlockSpec((r, c), lambda i: (i, 0))
    full = lambda a: pl.BlockSpec(a.shape, lambda i: (0,) * a.ndim)
    bf = jnp.bfloat16
    args = (ea128, hsab,
            We1.astype(bf), be1.reshape(1, -1), We2.astype(bf),
            be2.reshape(1, -1), bf1.reshape(1, -1),
            Wf1[:NSF].astype(bf), Wf2.astype(bf),
            bf2.reshape(1, -1).astype(bf),
            R.astype(bf), P.astype(bf), E16.astype(bf), E3.astype(bf), ones)
    in_specs = [blk(BLK, 128), blk(BLK, 128)] + [full(a) for a in args[2:]]
    return pl.pallas_call(
        _edge_body,
        grid=grid,
        in_specs=in_specs,
        out_specs=blk(BLK, TPW),
        out_shape=jax.ShapeDtypeStruct((N_EDGES, TPW), _F32),
    )(*args)


@functools.partial(
    pl.kernel,
    out_type=jax.ShapeDtypeStruct((2 * N_NODES, ACC_W), _F32),
    mesh=_MESH,
    scratch_types=[
        pltpu.VMEM((NCHUNK, CH), jnp.int32),
        pltpu.VMEM((ROWS_OUT, ACC_W), _F32),
        pltpu.VMEM_SHARED((N_NODES, ACC_W), _F32),
    ],
    compiler_params=pltpu.CompilerParams(use_tc_tiling_on_sc=False),
)
def _sc_scatter(tp_hbm, dst_hbm, zero_hbm, part_hbm, idx_d, rows, acc):
    c = lax.axis_index("c")
    s = lax.axis_index("s")
    wid = s * 2 + c
    base = wid * PER_TILE
    pltpu.sync_copy(zero_hbm.at[pl.ds(s * NPT, NPT)], acc.at[pl.ds(s * NPT, NPT)])
    pltpu.sync_copy(dst_hbm.at[wid], idx_d)
    plsc.subcore_barrier()

    def outer(j, carry):
        pltpu.sync_copy(
            tp_hbm.at[pl.ds(base + j * ROWS_OUT, ROWS_OUT), pl.ds(0, ACC_W)],
            rows)
        for b in range(KF):
            pltpu.sync_copy(rows.at[pl.ds(b * CH, CH)],
                            acc.at[idx_d.at[j * KF + b]], add=True)
        return carry

    lax.fori_loop(0, OUT_ITERS, outer, 0)
    plsc.subcore_barrier()
    pltpu.sync_copy(acc.at[pl.ds(s * NPT, NPT)],
                    part_hbm.at[pl.ds(c * N_NODES + s * NPT, NPT)])


def _combine_body(p_ref, out_ref):
    p = p_ref[0] + p_ref[1]
    cnt = p[:, 4 * NSF:4 * NSF + 1]
    out_ref[...] = p[:, :4 * NSF] / jnp.maximum(cnt, 1.0)


def _combine(parts):
    return pl.pallas_call(
        _combine_body,
        out_shape=jax.ShapeDtypeStruct((N_NODES, 4 * NSF), _F32),
    )(parts.reshape(2, N_NODES, ACC_W))


def kernel(x, edge_index, edge_attr, edge_sh, Wn1, bn1, Wn2, bn2,
           We1, be1, We2, be2, Wf1, bf1, Wf2, bf2):
    src = edge_index[0]
    dstv = edge_index[1]
    idx2 = jnp.stack([src, dstv + N_NODES], axis=1).reshape(NW, NCHUNK2, CH)
    dst = dstv.reshape(NW, NCHUNK, CH)
    R, P, E16, E3, ones = _consts()

    tab = _node_mlp(x, Wn1, bn1, Wn2, bn2, Wf1)
    hsab = _sc_gather(tab, idx2).reshape(N_EDGES, 128)
    ea128 = jnp.concatenate(
        [edge_attr, edge_sh[:, :4], jnp.zeros((N_EDGES, 44), _F32)], axis=1)
    tp = _edge_compute(ea128, hsab, We1, be1, We2, be2,
                       Wf1, bf1, Wf2, bf2, R, P, E16, E3, ones)
    zero = jnp.zeros((N_NODES, ACC_W), _F32)
    parts = _sc_scatter(tp, dst, zero)
    return _combine(parts)


# speedup vs baseline: 1.3110x; 1.2054x over previous
import functools

import jax
import jax.numpy as jnp
import numpy as np
from jax import lax
from jax.experimental import pallas as pl
from jax.experimental.pallas import tpu as pltpu
from jax.experimental.pallas import tpu_sc as plsc

NSF = 16
N_NODES = 10000
N_EDGES = 320000
TPW = 128
ACC_W = 80
NORM = 1.0 / np.sqrt(NSF)

NW = 32
PER_TILE = N_EDGES // NW
CH = 80
KF = 5
ROWS_OUT = CH * KF
OUT_ITERS = PER_TILE // ROWS_OUT
NCHUNK = PER_TILE // CH
NPT = N_NODES // 16


BLK = 3200

_F32 = jnp.float32


def _consts():
    p = np.arange(2 * NSF * NSF)
    R = np.zeros((NSF, 2 * NSF * NSF), np.float32)
    R[(p % 256) // NSF, p] = 1.0
    P = np.zeros((2 * NSF * NSF, 2 * NSF), np.float32)
    P[p, np.where(p < 256, 0, NSF) + p % NSF] = NORM
    E16 = np.zeros((NSF, 3 * NSF), np.float32)
    E3 = np.zeros((3, 3 * NSF), np.float32)
    for j in range(NSF):
        for c in range(3):
            E16[j, 3 * j + c] = 1.0
            E3[c, 3 * j + c] = 1.0
    ones = np.zeros((1, NSF), np.float32)
    ones[0, 0] = 1.0
    return (jnp.asarray(R), jnp.asarray(P), jnp.asarray(E16), jnp.asarray(E3),
            jnp.asarray(ones))


def _node_mlp_body(x_ref, w1_ref, b1_ref, w2_ref, b2_ref, ws_ref, wd_ref,
                   t_ref):
    t = jnp.maximum(
        jnp.dot(x_ref[...], w1_ref[...], preferred_element_type=_F32)
        + b1_ref[...], 0.0)
    h = (jnp.dot(t, w2_ref[...], preferred_element_type=_F32)
         + b2_ref[...])
    a = jnp.dot(h, ws_ref[...], preferred_element_type=_F32)
    b = jnp.dot(h, wd_ref[...], preferred_element_type=_F32)
    t_ref[:N_NODES, :] = jnp.concatenate([a, h], axis=-1)
    t_ref[N_NODES:, :] = jnp.concatenate(
        [b, jnp.zeros((N_NODES, NSF), _F32)], axis=-1)


def _node_mlp(x, Wn1, bn1, Wn2, bn2, Wf1):
    return pl.pallas_call(
        _node_mlp_body,
        out_shape=jax.ShapeDtypeStruct((2 * N_NODES, 4 * NSF), _F32),
    )(x, Wn1, bn1.reshape(1, -1), Wn2, bn2.reshape(1, -1),
      Wf1[NSF:2 * NSF], Wf1[2 * NSF:])


_MESH = plsc.VectorSubcoreMesh(core_axis_name="c", subcore_axis_name="s")


@functools.partial(
    pl.kernel,
    out_type=jax.ShapeDtypeStruct((N_EDGES, 128), _F32),
    mesh=_MESH,
    scratch_types=[
        pltpu.VMEM((NCHUNK, CH), jnp.int32),
        pltpu.VMEM((NCHUNK, CH), jnp.int32),
        pltpu.VMEM((ROWS_OUT, 4 * NSF), _F32),
        pltpu.VMEM((ROWS_OUT, 4 * NSF), _F32),
        pltpu.SemaphoreType.DMA,
        pltpu.SemaphoreType.DMA,
    ],
    compiler_params=pltpu.CompilerParams(use_tc_tiling_on_sc=False),
)
def _sc_gather(tab_hbm, src_hbm, dstn_hbm, hsab_hbm,
               idx_s, idx_d, st_s, st_d, sem_s, sem_d):
    c = lax.axis_index("c")
    s = lax.axis_index("s")
    wid = s * 2 + c
    base = wid * PER_TILE
    pltpu.sync_copy(src_hbm.at[wid], idx_s)
    pltpu.sync_copy(dstn_hbm.at[wid], idx_d)

    def outer(j, carry):
        cps = []
        for b in range(KF):
            k = j * KF + b
            cps.append(pltpu.async_copy(
                tab_hbm.at[idx_s.at[k]], st_s.at[pl.ds(b * CH, CH)], sem_s))
            cps.append(pltpu.async_copy(
                tab_hbm.at[idx_d.at[k]], st_d.at[pl.ds(b * CH, CH)], sem_d))
        for cp in cps:
            cp.wait()
        row0 = base + j * ROWS_OUT
        pltpu.sync_copy(st_s,
                        hsab_hbm.at[pl.ds(row0, ROWS_OUT), pl.ds(0, 4 * NSF)])
        pltpu.sync_copy(
            st_d, hsab_hbm.at[pl.ds(row0, ROWS_OUT), pl.ds(4 * NSF, 4 * NSF)])
        return carry

    lax.fori_loop(0, OUT_ITERS, outer, 0)


def _edge_body(ea_ref, hsab_ref,
               We1_ref, be1_ref, We2_ref, be2_ref,
               bf1_ref, Wf1e_ref, Wf2_ref, bf2_ref,
               R_ref, P_ref, E16_ref, E3_ref, ones_ref, tp_ref):
    pet = dict(preferred_element_type=_F32)
    bf = jnp.bfloat16
    ea = ea_ref[...]
    attr = ea[:, :80]
    sh = ea[:, 80:84]
    hsab = hsab_ref[...]
    a = hsab[:, : 3 * NSF]
    hs = hsab[:, 3 * NSF: 4 * NSF]
    b = hsab[:, 4 * NSF: 7 * NSF]
    t = jnp.maximum(
        jnp.dot(attr.astype(bf), We1_ref[...], **pet) + be1_ref[...], 0.0)
    e = jnp.dot(t.astype(bf), We2_ref[...], **pet) + be2_ref[...]
    g = jnp.maximum(
        jnp.dot(e.astype(bf), Wf1e_ref[...], **pet) + a + b + bf1_ref[...],
        0.0)
    w = (jnp.dot(g.astype(bf), Wf2_ref[...], **pet)
         + bf2_ref[...]).astype(bf)
    m = w * jnp.dot(hs.astype(bf), R_ref[...], **pet).astype(bf)
    sred = jnp.dot(m, P_ref[...], **pet)
    o0 = sred[:, :NSF] * sh[:, 0:1]
    o1 = (jnp.dot(sred[:, NSF:].astype(bf), E16_ref[...], **pet)
          * jnp.dot(sh[:, 1:4].astype(bf), E3_ref[...], **pet))
    tp_ref[:, : 4 * NSF] = jnp.concatenate([o0, o1], axis=-1)
    tp_ref[:, 4 * NSF: 5 * NSF] = jnp.broadcast_to(ones_ref[...], (BLK, NSF))


def _edge_compute(ea128, hsab, We1, be1, We2, be2,
                  Wf1, bf1, Wf2, bf2, R, P, E16, E3, ones):
    grid = (N_EDGES // BLK,)
    blk = lambda r, c: pl.BlockSpec((r, c), lambda i: (i, 0))
    full = lambda a: pl.BlockSpec(a.shape, lambda i: (0,) * a.ndim)
    bf = jnp.bfloat16
    args = (ea128, hsab,
            We1.astype(bf), be1.reshape(1, -1), We2.astype(bf),
            be2.reshape(1, -1), bf1.reshape(1, -1),
            Wf1[:NSF].astype(bf), Wf2.astype(bf),
            bf2.reshape(1, -1).astype(bf),
            R.astype(bf), P.astype(bf), E16.astype(bf), E3.astype(bf), ones)
    in_specs = [blk(BLK, 128), blk(BLK, 128)] + [full(a) for a in args[2:]]
    return pl.pallas_call(
        _edge_body,
        grid=grid,
        in_specs=in_specs,
        out_specs=blk(BLK, TPW),
        out_shape=jax.ShapeDtypeStruct((N_EDGES, TPW), _F32),
    )(*args)


@functools.partial(
    pl.kernel,
    out_type=jax.ShapeDtypeStruct((2 * N_NODES, ACC_W), _F32),
    mesh=_MESH,
    scratch_types=[
        pltpu.VMEM((NCHUNK, CH), jnp.int32),
        pltpu.VMEM((ROWS_OUT, ACC_W), _F32),
        pltpu.VMEM_SHARED((N_NODES, ACC_W), _F32),
    ],
    compiler_params=pltpu.CompilerParams(use_tc_tiling_on_sc=False),
)
def _sc_scatter(tp_hbm, dst_hbm, zero_hbm, part_hbm, idx_d, rows, acc):
    c = lax.axis_index("c")
    s = lax.axis_index("s")
    wid = s * 2 + c
    base = wid * PER_TILE
    pltpu.sync_copy(zero_hbm.at[pl.ds(s * NPT, NPT)], acc.at[pl.ds(s * NPT, NPT)])
    pltpu.sync_copy(dst_hbm.at[wid], idx_d)
    plsc.subcore_barrier()

    def outer(j, carry):
        pltpu.sync_copy(
            tp_hbm.at[pl.ds(base + j * ROWS_OUT, ROWS_OUT), pl.ds(0, ACC_W)],
            rows)
        for b in range(KF):
            pltpu.sync_copy(rows.at[pl.ds(b * CH, CH)],
                            acc.at[idx_d.at[j * KF + b]], add=True)
        return carry

    lax.fori_loop(0, OUT_ITERS, outer, 0)
    plsc.subcore_barrier()
    pltpu.sync_copy(acc.at[pl.ds(s * NPT, NPT)],
                    part_hbm.at[pl.ds(c * N_NODES + s * NPT, NPT)])


def _combine_body(p_ref, out_ref):
    p = p_ref[0] + p_ref[1]
    cnt = p[:, 4 * NSF:4 * NSF + 1]
    out_ref[...] = p[:, :4 * NSF] / jnp.maximum(cnt, 1.0)


def _combine(parts):
    return pl.pallas_call(
        _combine_body,
        out_shape=jax.ShapeDtypeStruct((N_NODES, 4 * NSF), _F32),
    )(parts.reshape(2, N_NODES, ACC_W))


def kernel(x, edge_index, edge_attr, edge_sh, Wn1, bn1, Wn2, bn2,
           We1, be1, We2, be2, Wf1, bf1, Wf2, bf2):
    src = edge_index[0].reshape(NW, NCHUNK, CH)
    dstn = (edge_index[1] + N_NODES).reshape(NW, NCHUNK, CH)
    dst = edge_index[1].reshape(NW, NCHUNK, CH)
    R, P, E16, E3, ones = _consts()

    tab = _node_mlp(x, Wn1, bn1, Wn2, bn2, Wf1)
    hsab = _sc_gather(tab, src, dstn)
    ea128 = jnp.concatenate(
        [edge_attr, edge_sh, jnp.zeros((N_EDGES, 39), _F32)], axis=1)
    tp = _edge_compute(ea128, hsab, We1, be1, We2, be2,
                       Wf1, bf1, Wf2, bf2, R, P, E16, E3, ones)
    zero = jnp.zeros((N_NODES, ACC_W), _F32)
    parts = _sc_scatter(tp, dst, zero)
    return _combine(parts)


# speedup vs baseline: 1.3896x; 1.0600x over previous
import functools

import jax
import jax.numpy as jnp
import numpy as np
from jax import lax
from jax.experimental import pallas as pl
from jax.experimental.pallas import tpu as pltpu
from jax.experimental.pallas import tpu_sc as plsc

NSF = 16
N_NODES = 10000
N_EDGES = 320000
TPW = 128
ACC_W = 80
NORM = 1.0 / np.sqrt(NSF)

NW = 32
PER_TILE = N_EDGES // NW
CH = 80
KF = 5
ROWS_OUT = CH * KF
OUT_ITERS = PER_TILE // ROWS_OUT
NCHUNK = PER_TILE // CH
NPT = N_NODES // 16


BLK = 3200

_F32 = jnp.float32


def _consts():
    p = np.arange(2 * NSF * NSF)
    R = np.zeros((NSF, 2 * NSF * NSF), np.float32)
    R[(p % 256) // NSF, p] = 1.0
    P = np.zeros((2 * NSF * NSF, 2 * NSF), np.float32)
    P[p, np.where(p < 256, 0, NSF) + p % NSF] = NORM
    E16 = np.zeros((NSF, 3 * NSF), np.float32)
    E3 = np.zeros((3, 3 * NSF), np.float32)
    for j in range(NSF):
        for c in range(3):
            E16[j, 3 * j + c] = 1.0
            E3[c, 3 * j + c] = 1.0
    ones = np.zeros((1, NSF), np.float32)
    ones[0, 0] = 1.0
    return (jnp.asarray(R), jnp.asarray(P), jnp.asarray(E16), jnp.asarray(E3),
            jnp.asarray(ones))


def _node_mlp_body(x_ref, w1_ref, b1_ref, w2_ref, b2_ref, ws_ref, wd_ref,
                   t_ref):
    t = jnp.maximum(
        jnp.dot(x_ref[...], w1_ref[...], preferred_element_type=_F32)
        + b1_ref[...], 0.0)
    h = (jnp.dot(t, w2_ref[...], preferred_element_type=_F32)
         + b2_ref[...])
    a = jnp.dot(h, ws_ref[...], preferred_element_type=_F32)
    b = jnp.dot(h, wd_ref[...], preferred_element_type=_F32)
    t_ref[:N_NODES, :] = jnp.concatenate([a, h], axis=-1)
    t_ref[N_NODES:, :] = jnp.concatenate(
        [b, jnp.zeros((N_NODES, NSF), _F32)], axis=-1)


def _node_mlp(x, Wn1, bn1, Wn2, bn2, Wf1):
    return pl.pallas_call(
        _node_mlp_body,
        out_shape=jax.ShapeDtypeStruct((2 * N_NODES, 4 * NSF), _F32),
    )(x, Wn1, bn1.reshape(1, -1), Wn2, bn2.reshape(1, -1),
      Wf1[NSF:2 * NSF], Wf1[2 * NSF:])


_MESH = plsc.VectorSubcoreMesh(core_axis_name="c", subcore_axis_name="s")


@functools.partial(
    pl.kernel,
    out_type=jax.ShapeDtypeStruct((N_EDGES, 128), _F32),
    mesh=_MESH,
    scratch_types=[
        pltpu.VMEM((NCHUNK, CH), jnp.int32),
        pltpu.VMEM((NCHUNK, CH), jnp.int32),
        pltpu.VMEM((ROWS_OUT, 4 * NSF), _F32),
        pltpu.VMEM((ROWS_OUT, 4 * NSF), _F32),
        pltpu.SemaphoreType.DMA,
        pltpu.SemaphoreType.DMA,
    ],
    compiler_params=pltpu.CompilerParams(use_tc_tiling_on_sc=False),
)
def _sc_gather(tab_hbm, src_hbm, dstn_hbm, hsab_hbm,
               idx_s, idx_d, st_s, st_d, sem_s, sem_d):
    c = lax.axis_index("c")
    s = lax.axis_index("s")
    wid = s * 2 + c
    base = wid * PER_TILE
    pltpu.sync_copy(src_hbm.at[wid], idx_s)
    pltpu.sync_copy(dstn_hbm.at[wid], idx_d)

    def outer(j, carry):
        cps = []
        for b in range(KF):
            k = j * KF + b
            cps.append(pltpu.async_copy(
                tab_hbm.at[idx_s.at[k]], st_s.at[pl.ds(b * CH, CH)], sem_s))
            cps.append(pltpu.async_copy(
                tab_hbm.at[idx_d.at[k]], st_d.at[pl.ds(b * CH, CH)], sem_d))
        for cp in cps:
            cp.wait()
        row0 = base + j * ROWS_OUT
        pltpu.sync_copy(st_s,
                        hsab_hbm.at[pl.ds(row0, ROWS_OUT), pl.ds(0, 4 * NSF)])
        pltpu.sync_copy(
            st_d, hsab_hbm.at[pl.ds(row0, ROWS_OUT), pl.ds(4 * NSF, 4 * NSF)])
        return carry

    lax.fori_loop(0, OUT_ITERS, outer, 0)


def _edge_body(ea_ref, hsab_ref,
               We1_ref, be1_ref, We2_ref, be2_ref,
               bf1_ref, Wf1e_ref, Wf2_ref, bf2_ref,
               R_ref, P_ref, E16_ref, E3_ref, ones_ref, tp_ref):
    pet = dict(preferred_element_type=_F32)
    bf = jnp.bfloat16
    ea = ea_ref[...]
    attr = ea[:, :80]
    sh = ea[:, 80:84].astype(_F32)
    hsab = hsab_ref[...]
    a = hsab[:, : 3 * NSF]
    hs = hsab[:, 3 * NSF: 4 * NSF]
    b = hsab[:, 4 * NSF: 7 * NSF]
    t = jnp.maximum(
        jnp.dot(attr, We1_ref[...], **pet) + be1_ref[...], 0.0)
    e = jnp.dot(t.astype(bf), We2_ref[...], **pet) + be2_ref[...]
    g = jnp.maximum(
        jnp.dot(e.astype(bf), Wf1e_ref[...], **pet) + a + b + bf1_ref[...],
        0.0)
    w = (jnp.dot(g.astype(bf), Wf2_ref[...], **pet)
         + bf2_ref[...]).astype(bf)
    m = w * jnp.dot(hs.astype(bf), R_ref[...], **pet).astype(bf)
    sred = jnp.dot(m, P_ref[...], **pet)
    o0 = sred[:, :NSF] * sh[:, 0:1]
    o1 = (jnp.dot(sred[:, NSF:].astype(bf), E16_ref[...], **pet)
          * jnp.dot(ea[:, 81:84], E3_ref[...], **pet))
    tp_ref[:, : 4 * NSF] = jnp.concatenate([o0, o1], axis=-1)
    tp_ref[:, 4 * NSF: 5 * NSF] = jnp.broadcast_to(ones_ref[...], (BLK, NSF))


def _edge_compute(ea128, hsab, We1, be1, We2, be2,
                  Wf1, bf1, Wf2, bf2, R, P, E16, E3, ones):
    grid = (N_EDGES // BLK,)
    blk = lambda r, c: pl.BlockSpec((r, c), lambda i: (i, 0))
    full = lambda a: pl.BlockSpec(a.shape, lambda i: (0,) * a.ndim)
    bf = jnp.bfloat16
    args = (ea128, hsab,
            We1.astype(bf), be1.reshape(1, -1), We2.astype(bf),
            be2.reshape(1, -1), bf1.reshape(1, -1),
            Wf1[:NSF].astype(bf), Wf2.astype(bf),
            bf2.reshape(1, -1).astype(bf),
            R.astype(bf), P.astype(bf), E16.astype(bf), E3.astype(bf), ones)
    in_specs = [blk(BLK, 128), blk(BLK, 128)] + [full(a) for a in args[2:]]
    return pl.pallas_call(
        _edge_body,
        grid=grid,
        in_specs=in_specs,
        out_specs=blk(BLK, TPW),
        out_shape=jax.ShapeDtypeStruct((N_EDGES, TPW), _F32),
    )(*args)


@functools.partial(
    pl.kernel,
    out_type=jax.ShapeDtypeStruct((2 * N_NODES, ACC_W), _F32),
    mesh=_MESH,
    scratch_types=[
        pltpu.VMEM((NCHUNK, CH), jnp.int32),
        pltpu.VMEM((ROWS_OUT, ACC_W), _F32),
        pltpu.VMEM_SHARED((N_NODES, ACC_W), _F32),
    ],
    compiler_params=pltpu.CompilerParams(use_tc_tiling_on_sc=False),
)
def _sc_scatter(tp_hbm, dst_hbm, zero_hbm, part_hbm, idx_d, rows, acc):
    c = lax.axis_index("c")
    s = lax.axis_index("s")
    wid = s * 2 + c
    base = wid * PER_TILE
    pltpu.sync_copy(zero_hbm.at[pl.ds(s * NPT, NPT)], acc.at[pl.ds(s * NPT, NPT)])
    pltpu.sync_copy(dst_hbm.at[wid], idx_d)
    plsc.subcore_barrier()

    def outer(j, carry):
        pltpu.sync_copy(
            tp_hbm.at[pl.ds(base + j * ROWS_OUT, ROWS_OUT), pl.ds(0, ACC_W)],
            rows)
        for b in range(KF):
            pltpu.sync_copy(rows.at[pl.ds(b * CH, CH)],
                            acc.at[idx_d.at[j * KF + b]], add=True)
        return carry

    lax.fori_loop(0, OUT_ITERS, outer, 0)
    plsc.subcore_barrier()
    pltpu.sync_copy(acc.at[pl.ds(s * NPT, NPT)],
                    part_hbm.at[pl.ds(c * N_NODES + s * NPT, NPT)])


def _combine_body(p_ref, out_ref):
    p = p_ref[0] + p_ref[1]
    cnt = p[:, 4 * NSF:4 * NSF + 1]
    out_ref[...] = p[:, :4 * NSF] / jnp.maximum(cnt, 1.0)


def _combine(parts):
    return pl.pallas_call(
        _combine_body,
        out_shape=jax.ShapeDtypeStruct((N_NODES, 4 * NSF), _F32),
    )(parts.reshape(2, N_NODES, ACC_W))


def kernel(x, edge_index, edge_attr, edge_sh, Wn1, bn1, Wn2, bn2,
           We1, be1, We2, be2, Wf1, bf1, Wf2, bf2):
    src = edge_index[0].reshape(NW, NCHUNK, CH)
    dstn = (edge_index[1] + N_NODES).reshape(NW, NCHUNK, CH)
    dst = edge_index[1].reshape(NW, NCHUNK, CH)
    R, P, E16, E3, ones = _consts()

    tab = _node_mlp(x, Wn1, bn1, Wn2, bn2, Wf1)
    hsab = _sc_gather(tab, src, dstn)
    ea128 = jnp.concatenate(
        [edge_attr, edge_sh, jnp.zeros((N_EDGES, 39), _F32)],
        axis=1).astype(jnp.bfloat16)
    tp = _edge_compute(ea128, hsab, We1, be1, We2, be2,
                       Wf1, bf1, Wf2, bf2, R, P, E16, E3, ones)
    zero = jnp.zeros((N_NODES, ACC_W), _F32)
    parts = _sc_scatter(tp, dst, zero)
    return _combine(parts)
